# baseline (speedup 1.0000x reference)
import jax
import jax.numpy as jnp
from jax import lax
from jax.experimental import pallas as pl
from jax.experimental.pallas import tpu as pltpu
from jax.experimental.pallas import tpu_sc as plsc

N_USERS = 25000
N_ITEMS = 25000
N_NODES = 50000
D = 64
H = D // 2
N_LAYERS = 3
N_EDGES = 800000

NC = 2
NS = 16
NP = 50176
NT = NP // NS
ROWS_PT = 400
EP = ROWS_PT * 128 * NS
JC = 8
NSEG = 5
SROWS = ROWS_PT // NSEG
SPAIR = SROWS // 16
FROWS = 784
MROWS = 196
ZROWS = 196


def _body(x0, src2d, dst2d, xl,
          rows_v, rows_w, sblk, dblk, ones_v, zbuf, recip_v,
          sem_g, sem_s, sem_z, cnt_sp, acc_sp):
    c = lax.axis_index("c")
    s = lax.axis_index("s")
    node0 = s * NT
    erow0 = s * ROWS_PT

    def _ones_row(i, _):
        ones_v[pl.ds(i * 16, 16)] = jnp.ones((16,), jnp.float32)
        return 0
    lax.fori_loop(0, 128 // 16, _ones_row, 0)

    def _zb_row(i, _):
        zbuf[i, pl.ds(0, 32)] = jnp.zeros((32,), jnp.bfloat16)
        return 0
    lax.fori_loop(0, ZROWS, _zb_row, 0)

    def _zr_row(i, _):
        recip_v[pl.ds(i * 16, 16)] = jnp.zeros((16,), jnp.float32)
        return 0
    lax.fori_loop(0, NT // 16, _zr_row, 0)

    pltpu.sync_copy(recip_v, cnt_sp.at[pl.ds(node0, NT)])
    zds = [pltpu.async_copy(zbuf, acc_sp.at[pl.ds(node0 + i * ZROWS, ZROWS)],
                            sem_z) for i in range(NT // ZROWS)]
    for d_ in zds:
        d_.wait()
    plsc.subcore_barrier()

    def _edge_pass(xinb, with_counts=False):
        def _fire_g(kbase, rows):
            for j in range(JC):
                pltpu.async_copy(xinb.at[sblk.at[kbase + j]],
                                 rows.at[pl.ds(j * 128, 128)], sem_g)

        def _drain_g(kbase, rows):
            for j in range(JC):
                pltpu.make_async_copy(
                    xinb.at[sblk.at[kbase + j]],
                    rows.at[pl.ds(j * 128, 128)], sem_g).wait()

        def _scat(kbase, rows):
            descs = [pltpu.async_copy(rows.at[pl.ds(j * 128, 128)],
                                      acc_sp.at[dblk.at[kbase + j]],
                                      sem_s, add=True)
                     for j in range(JC)]
            if with_counts:
                descs += [pltpu.async_copy(ones_v, cnt_sp.at[dblk.at[kbase + j]],
                                           sem_z, add=True)
                          for j in range(JC)]
            for d_ in descs:
                d_.wait()

        def _seg(sg, _):
            r0 = erow0 + sg * SROWS
            pltpu.sync_copy(src2d.at[pl.ds(r0, SROWS)], sblk)
            pltpu.sync_copy(dst2d.at[pl.ds(r0, SROWS)], dblk)
            _fire_g(0, rows_v)

            def _pair(k, _):
                _drain_g(k * 16, rows_v)
                _fire_g(k * 16 + JC, rows_w)
                _scat(k * 16, rows_v)

                @pl.when(k < SPAIR - 1)
                def _():
                    _fire_g(k * 16 + 16, rows_v)
                _drain_g(k * 16 + JC, rows_w)
                _scat(k * 16 + JC, rows_w)
                return 0
            lax.fori_loop(0, SPAIR, _pair, 0)
            return 0
        lax.fori_loop(0, NSEG, _seg, 0)

    rbufs = (rows_v, rows_w)

    def _finalize(xoutb):
        def _fin_read(i):
            return pltpu.async_copy(
                acc_sp.at[pl.ds(node0 + i * FROWS, FROWS)],
                rbufs[i % 2].at[pl.ds(0, FROWS)], sem_g)

        def _fin_compute(i, buf):
            def _blk(b, _):
                rvec = recip_v[pl.ds(i * FROWS + b * 16, 16)]
                for k in range(16):
                    r = b * 16 + k
                    lo = buf[r, pl.ds(0, 16)].astype(jnp.float32) * rvec[k]
                    hi = buf[r, pl.ds(16, 16)].astype(jnp.float32) * rvec[k]
                    buf[r, pl.ds(0, 16)] = lo.astype(jnp.bfloat16)
                    buf[r, pl.ds(16, 16)] = hi.astype(jnp.bfloat16)
                return 0
            lax.fori_loop(0, FROWS // 16, _blk, 0)

        nfc = NT // FROWS
        zds2 = []
        wr = [None] * nfc
        rd = _fin_read(0)
        for i in range(nfc):
            rd.wait()
            if i + 1 < nfc:
                if i >= 1:
                    wr[i - 1].wait()
                rd = _fin_read(i + 1)
            for z4 in range(FROWS // ZROWS):
                zds2.append(pltpu.async_copy(
                    zbuf,
                    acc_sp.at[pl.ds(node0 + i * FROWS + z4 * ZROWS, ZROWS)],
                    sem_z))
            _fin_compute(i, rbufs[i % 2])
            wr[i] = pltpu.async_copy(
                rbufs[i % 2].at[pl.ds(0, FROWS)],
                xoutb.at[pl.ds(node0 + i * FROWS, FROWS)], sem_s)
        for d_ in zds2 + [wr[nfc - 2], wr[nfc - 1]]:
            d_.wait()

    _edge_pass(x0.at[c], with_counts=True)
    plsc.subcore_barrier()

    pltpu.sync_copy(cnt_sp.at[pl.ds(node0, NT)], recip_v)

    def _recip_blk(i, _):
        v = recip_v[pl.ds(i * 16, 16)]
        recip_v[pl.ds(i * 16, 16)] = 1.0 / jnp.maximum(v, 1.0)
        return 0
    lax.fori_loop(0, NT // 16, _recip_blk, 0)

    _finalize(xl.at[c])
    plsc.subcore_barrier()

    def _passl(l2, _):
        _edge_pass(xl.at[l2 * 2 + c])
        plsc.subcore_barrier()
        _finalize(xl.at[(l2 + 1) * 2 + c])
        plsc.subcore_barrier()
        return 0
    lax.fori_loop(0, N_LAYERS - 1, _passl, 0)


_sgl_kernel = pl.kernel(
    _body,
    out_type=jax.ShapeDtypeStruct((2 * N_LAYERS, NP, H), jnp.bfloat16),
    mesh=plsc.VectorSubcoreMesh(core_axis_name="c", subcore_axis_name="s",
                                num_cores=NC, num_subcores=NS),
    compiler_params=pltpu.CompilerParams(use_tc_tiling_on_sc=False),
    scratch_types=(
        pltpu.VMEM((JC * 128, H), jnp.bfloat16),
        pltpu.VMEM((JC * 128, H), jnp.bfloat16),
        pltpu.VMEM((SROWS, 128), jnp.int32),
        pltpu.VMEM((SROWS, 128), jnp.int32),
        pltpu.VMEM((128,), jnp.float32),
        pltpu.VMEM((ZROWS, H), jnp.bfloat16),
        pltpu.VMEM((NT,), jnp.float32),
        pltpu.SemaphoreType.DMA,
        pltpu.SemaphoreType.DMA,
        pltpu.SemaphoreType.DMA,
        pltpu.VMEM_SHARED((NP,), jnp.float32),
        pltpu.VMEM_SHARED((NP, H), jnp.bfloat16),
    ),
)


def _mean_body(x0_ref, xl_ref, o_ref):
    for c_h in range(2):
        acc = (x0_ref[c_h].astype(jnp.float32)
               + xl_ref[c_h].astype(jnp.float32)
               + xl_ref[2 + c_h].astype(jnp.float32)
               + xl_ref[4 + c_h].astype(jnp.float32))
        o_ref[:, c_h * H:(c_h + 1) * H] = acc * 0.25


MB = 3136

_mean_kernel = pl.pallas_call(
    _mean_body,
    out_shape=jax.ShapeDtypeStruct((NP, D), jnp.float32),
    grid=(NP // MB,),
    in_specs=[
        pl.BlockSpec((2, MB, H), lambda i: (0, i, 0)),
        pl.BlockSpec((2 * N_LAYERS, MB, H), lambda i: (0, i, 0)),
    ],
    out_specs=pl.BlockSpec((MB, D), lambda i: (i, 0)),
)


@jax.jit
def kernel(user_table, item_table, edge_index):
    x = jnp.concatenate([user_table, item_table], axis=0)
    x = jnp.pad(x, ((0, NP - N_NODES), (0, 0)))
    xb = x.astype(jnp.bfloat16)
    x0 = jnp.stack([xb[:, 0:32], xb[:, 32:64]], axis=0)
    src = jnp.pad(edge_index[0], (0, EP - N_EDGES))
    dst = jnp.pad(edge_index[1], (0, EP - N_EDGES), constant_values=NP - 1)
    src2d = src.reshape(NS * ROWS_PT, 128)
    dst2d = dst.reshape(NS * ROWS_PT, 128)
    xl = _sgl_kernel(x0, src2d, dst2d)
    full = _mean_kernel(x0, xl)
    return full[:N_USERS], full[N_USERS:N_NODES]
